# Initial kernel scaffold; baseline (speedup 1.0000x reference)
import jax
import jax.numpy as jnp
from jax.experimental import pallas as pl


def kernel(x, edge_index, W, b):
    raise NotImplementedError("write your pallas kernel here")


# speedup vs baseline: 4.8310x; 4.8310x over previous
import functools

import jax
import jax.numpy as jnp
from jax import lax
from jax.experimental import pallas as pl
from jax.experimental.pallas import tpu as pltpu
from jax.experimental.pallas import tpu_sc as plsc

N_NODES = 10000
N_EDGES = 320000
D = 128

NC = 2
NS = 16
NW = NC * NS

CH = 128
NCH = -(-N_EDGES // (NW * CH))
EDGES_PER_TILE = NCH * CH
E_PAD = NW * EDGES_PER_TILE

ACC_PT = (-(-(N_NODES + 1) // NS) + 127) // 128 * 128
ACC_N = ACC_PT * NS


def _sc_body(x_hbm, srci_hbm, dsti_hbm, z_hbm, out_hbm,
             idx_s_v, idx_d_v, rows_v, dsem, accum):
    c = lax.axis_index("c")
    s = lax.axis_index("s")
    wid = s * NC + c

    pltpu.sync_copy(z_hbm, accum.at[pl.ds(s * ACC_PT, ACC_PT)])
    pltpu.sync_copy(srci_hbm.at[wid], idx_s_v)
    pltpu.sync_copy(dsti_hbm.at[wid], idx_d_v)
    plsc.subcore_barrier()

    def chunk(j, carry):
        pltpu.async_copy(x_hbm.at[idx_s_v.at[j]], rows_v, dsem).wait()
        pltpu.sync_copy(rows_v, accum.at[idx_d_v.at[j]], add=True)
        return carry

    lax.fori_loop(0, NCH, chunk, 0)
    plsc.subcore_barrier()

    pltpu.sync_copy(accum.at[pl.ds(s * ACC_PT, ACC_PT)],
                    out_hbm.at[c, pl.ds(s * ACC_PT, ACC_PT)])


_sc_call = pl.kernel(
    _sc_body,
    out_type=jax.ShapeDtypeStruct((NC, ACC_N, D), jnp.float32),
    mesh=plsc.VectorSubcoreMesh(core_axis_name="c", subcore_axis_name="s",
                                num_cores=NC, num_subcores=NS),
    scratch_types=[
        pltpu.VMEM((NCH, CH), jnp.int32),
        pltpu.VMEM((NCH, CH), jnp.int32),
        pltpu.VMEM((CH, D), jnp.float32),
        pltpu.SemaphoreType.DMA,
        pltpu.VMEM_SHARED((ACC_N, D), jnp.float32),
    ],
)


def _tc_body(p_ref, x_ref, w_ref, b_ref, hagg_ref, emb_ref):
    hagg_ref[...] = p_ref[0] + p_ref[1]
    emb_ref[...] = lax.dot_general(
        x_ref[...], w_ref[...], (((1,), (1,)), ((), ())),
        preferred_element_type=jnp.float32) + b_ref[...]


BLK = 1000


def _tc_call(partials, x, W, b2):
    return pl.pallas_call(
        _tc_body,
        grid=(N_NODES // BLK,),
        in_specs=[
            pl.BlockSpec((NC, BLK, D), lambda i: (0, i, 0)),
            pl.BlockSpec((BLK, D), lambda i: (i, 0)),
            pl.BlockSpec((D, D), lambda i: (0, 0)),
            pl.BlockSpec((1, D), lambda i: (0, 0)),
        ],
        out_specs=[pl.BlockSpec((BLK, D), lambda i: (i, 0)),
                   pl.BlockSpec((BLK, D), lambda i: (i, 0))],
        out_shape=[jax.ShapeDtypeStruct((N_NODES, D), jnp.float32),
                   jax.ShapeDtypeStruct((N_NODES, D), jnp.float32)],
    )(partials, x, W, b2)


@jax.jit
def kernel(x, edge_index, W, b):
    src = edge_index[0].astype(jnp.int32)
    dst = edge_index[1].astype(jnp.int32)
    pad = E_PAD - N_EDGES
    src_p = jnp.concatenate([src, jnp.zeros((pad,), jnp.int32)]).reshape(NW, NCH, CH)
    dst_p = jnp.concatenate([dst, jnp.full((pad,), N_NODES, jnp.int32)]).reshape(NW, NCH, CH)
    z = jnp.zeros((ACC_PT, D), jnp.float32)
    partials = _sc_call(x, src_p, dst_p, z)
    h_agg, emb = _tc_call(partials, x, W, b.reshape(1, D))
    return (h_agg, emb)
